# baseline (speedup 1.0000x reference)
import functools

import jax
import jax.numpy as jnp
from jax import lax
from jax.experimental import pallas as pl
from jax.experimental.pallas import tpu as pltpu
from jax.experimental.pallas import tpu_sc as plsc

_NSAMP = 16384
_V = 100000
_NPAD = 100352
_CHUNK = _NPAD // 16
_SEG = _CHUNK // 16
_TAIL = _V - 15 * _CHUNK
_SAMP_W = _NSAMP // 32
_GRPS = _SAMP_W // 16
_UNROLL_G = 4
_UNROLL_K = 8


def _lane_gather(v, idx):
    dnums = lax.GatherDimensionNumbers(
        offset_dims=(), collapsed_slice_dims=(0,), start_index_map=(0,))
    return lax.gather(v, idx[:, None], dnums, slice_sizes=(1,),
                      mode=lax.GatherScatterMode.PROMISE_IN_BOUNDS)


def _lane_prefix(v, lanes):
    for k in (1, 2, 4, 8):
        sh = _lane_gather(v, jnp.maximum(lanes - k, 0))
        v = v + jnp.where(lanes >= k, sh, jnp.float32(0.0))
    return v


def _sampler(freqs_hbm, u_hbm, out_hbm,
             cdf_v, fr_v, u_v, o_v, bnd_v, off_v, segb_v, cdf_sh):
    c_id = lax.axis_index("c")
    s_id = lax.axis_index("s")
    wid = s_id * 2 + c_id
    base = s_id * _CHUNK
    lanes = lax.iota(jnp.int32, 16)

    @pl.when(s_id == 15)
    def _():
        pltpu.sync_copy(freqs_hbm.at[pl.ds(base, _TAIL)], fr_v.at[pl.ds(0, _TAIL)])
        for z in range((_CHUNK - _TAIL) // 16):
            fr_v[pl.ds(_TAIL + z * 16, 16)] = jnp.zeros((16,), jnp.float32)

    @pl.when(s_id != 15)
    def _():
        pltpu.sync_copy(freqs_hbm.at[pl.ds(base, _CHUNK)], fr_v)

    pltpu.sync_copy(u_hbm.at[pl.ds(wid * _SAMP_W, _SAMP_W)], u_v)

    with jax.named_scope("phase_cumsum"):
        rd_base = lanes * _SEG
        wr_base = base + rd_base

        def pass_a(i, acc):
            for d in range(_UNROLL_K):
                k = i * _UNROLL_K + d
                acc = acc + plsc.load_gather(fr_v, [rd_base + k])
                plsc.store_scatter(cdf_v, [wr_base + k], acc)
            return acc

        lax.fori_loop(0, _SEG // _UNROLL_K, pass_a,
                      jnp.zeros((16,), jnp.float32))

    with jax.named_scope("phase_exchange"):
        pltpu.sync_copy(cdf_v.at[pl.ds(base, _CHUNK)],
                        cdf_sh.at[pl.ds(base, _CHUNK)])
        plsc.subcore_barrier()
        pltpu.sync_copy(cdf_sh, cdf_v)

    with jax.named_scope("phase_tables"):
        seg_ends = lanes * _SEG + (_SEG - 1)
        for r in range(16):
            srow = plsc.load_gather(cdf_v, [r * _CHUNK + seg_ends])
            segb_v[r] = _lane_prefix(srow, lanes)
        tot_c = plsc.load_gather(segb_v, [lanes, jnp.full((16,), 15, jnp.int32)])
        inc = _lane_prefix(tot_c, lanes)
        bnd_v[...] = inc
        off_v[...] = inc - tot_c
        total = _lane_gather(inc, jnp.full((16,), 15, jnp.int32))

    def search16(t):
        c = jnp.zeros((16,), jnp.int32)
        for b in (8, 4, 2, 1):
            val = plsc.load_gather(bnd_v, [c + (b - 1)])
            c = c + jnp.where(val < t, b, 0)
        c = jnp.minimum(c, 15)
        t2 = t - plsc.load_gather(off_v, [c])
        s = jnp.zeros((16,), jnp.int32)
        for b in (8, 4, 2, 1):
            val = plsc.load_gather(segb_v, [c, s + (b - 1)])
            s = s + jnp.where(val < t2, b, 0)
        s = jnp.minimum(s, 15)
        sexcl = plsc.load_gather(segb_v, [c, jnp.maximum(s - 1, 0)])
        t3 = t2 - jnp.where(s > 0, sexcl, jnp.float32(0.0))
        segstart = c * _CHUNK + s * _SEG
        lo = jnp.zeros((16,), jnp.int32)
        for b in (256, 128, 64, 32, 16, 8, 4, 2, 1):
            probe = jnp.minimum(lo + (b - 1), _SEG - 1)
            val = plsc.load_gather(cdf_v, [segstart + probe])
            lo = lo + jnp.where(val < t3, b, 0)
        return jnp.minimum(segstart + lo, _V - 1)

    with jax.named_scope("phase_search"):
        def grp_body(g, _):
            for d in range(_UNROLL_G):
                gg = g * _UNROLL_G + d
                t = u_v[pl.ds(gg * 16, 16)] * total
                idx = search16(t)
                o_v[pl.ds(gg * 16, 16)] = idx.astype(jnp.float32)
            return 0

        lax.fori_loop(0, _GRPS // _UNROLL_G, grp_body, 0)

    pltpu.sync_copy(o_v, out_hbm.at[pl.ds(wid * _SAMP_W, _SAMP_W)])


def kernel(data, freqs):
    del data
    u = jax.random.uniform(jax.random.key(42), (_NSAMP,), dtype=jnp.float32)
    mesh = plsc.VectorSubcoreMesh(core_axis_name="c", subcore_axis_name="s")
    run = pl.kernel(
        _sampler,
        mesh=mesh,
        compiler_params=pltpu.CompilerParams(needs_layout_passes=False),
        out_type=jax.ShapeDtypeStruct((_NSAMP,), jnp.float32),
        scratch_types=[
            pltpu.VMEM((_NPAD,), jnp.float32),
            pltpu.VMEM((_CHUNK,), jnp.float32),
            pltpu.VMEM((_SAMP_W,), jnp.float32),
            pltpu.VMEM((_SAMP_W,), jnp.float32),
            pltpu.VMEM((16,), jnp.float32),
            pltpu.VMEM((16,), jnp.float32),
            pltpu.VMEM((16, 16), jnp.float32),
            pltpu.VMEM_SHARED((_NPAD,), jnp.float32),
        ],
    )
    return run(freqs, u)


# speedup vs baseline: 1.0147x; 1.0147x over previous
import functools

import numpy as np
import jax
import jax.numpy as jnp
from jax import lax
from jax.experimental import pallas as pl
from jax.experimental.pallas import tpu as pltpu
from jax.experimental.pallas import tpu_sc as plsc

_NSAMP = 16384
_V = 100000
_NPAD = 100352
_CHUNK = _NPAD // 16
_SEG = _CHUNK // 16
_TAIL = _V - 15 * _CHUNK
_SAMP_W = _NSAMP // 32
_GRPS = _SAMP_W // 16
_UNROLL_G = 4
_UNROLL_K = 8

def _lane_gather(v, idx):
    dnums = lax.GatherDimensionNumbers(
        offset_dims=(), collapsed_slice_dims=(0,), start_index_map=(0,))
    return lax.gather(v, idx[:, None], dnums, slice_sizes=(1,),
                      mode=lax.GatherScatterMode.PROMISE_IN_BOUNDS)


def _lane_prefix(v, lanes):
    for k in (1, 2, 4, 8):
        sh = _lane_gather(v, jnp.maximum(lanes - k, 0))
        v = v + jnp.where(lanes >= k, sh, jnp.float32(0.0))
    return v


def _sampler(freqs_hbm, u_hbm, out_hbm,
             cdf_v, fr_v, u_v, o_v, gbl_v, cdf_sh, sem_f, sem_u):
    c_id = lax.axis_index("c")
    s_id = lax.axis_index("s")
    wid = s_id * 2 + c_id
    base = s_id * _CHUNK
    lanes = lax.iota(jnp.int32, 16)

    h_f = pltpu.async_copy(freqs_hbm.at[pl.ds(base, _TAIL)],
                           fr_v.at[pl.ds(0, _TAIL)], sem_f)
    h_u = pltpu.async_copy(u_hbm.at[pl.ds(wid * _SAMP_W, _SAMP_W)], u_v, sem_u)

    @pl.when(s_id == 15)
    def _():
        for z in range((_CHUNK - _TAIL) // 16):
            fr_v[pl.ds(_TAIL + z * 16, 16)] = jnp.zeros((16,), jnp.float32)

    @pl.when(s_id != 15)
    def _():
        pltpu.sync_copy(freqs_hbm.at[pl.ds(base + _TAIL, _CHUNK - _TAIL)],
                        fr_v.at[pl.ds(_TAIL, _CHUNK - _TAIL)])

    h_f.wait()

    with jax.named_scope("phase_cumsum"):
        rd_base = lanes * _SEG

        def pass_a(i, acc):
            for d in range(_UNROLL_K):
                k = i * _UNROLL_K + d
                acc = acc + plsc.load_gather(fr_v, [rd_base + k])
                cdf_v[pl.ds(base + k * 16, 16)] = acc
            return acc

        lax.fori_loop(0, _SEG // _UNROLL_K, pass_a,
                      jnp.zeros((16,), jnp.float32))

    with jax.named_scope("phase_exchange"):
        pltpu.sync_copy(cdf_v.at[pl.ds(base, _CHUNK)],
                        cdf_sh.at[pl.ds(base, _CHUNK)])
        plsc.subcore_barrier()
        pltpu.sync_copy(cdf_sh, cdf_v)

    with jax.named_scope("phase_tables"):
        for r in range(16):
            srow = cdf_v[pl.ds(r * _CHUNK + (_SEG - 1) * 16, 16)]
            gbl_v[pl.ds(r * 16, 16)] = _lane_prefix(srow, lanes)
        tot_c = plsc.load_gather(gbl_v, [lanes * 16 + 15])
        bnd = _lane_prefix(tot_c, lanes)
        off = bnd - tot_c
        total = _lane_gather(bnd, jnp.full((16,), 15, jnp.int32))
        for r in range(16):
            offr = _lane_gather(off, jnp.full((16,), r, jnp.int32))
            gbl_v[pl.ds(r * 16, 16)] = gbl_v[pl.ds(r * 16, 16)] + offr

    h_u.wait()

    def search16(t):
        j = jnp.zeros((16,), jnp.int32)
        for b in (128, 64, 32, 16, 8, 4, 2, 1):
            val = plsc.load_gather(gbl_v, [j + (b - 1)])
            j = j + jnp.where(val < t, b, 0)
        j = jnp.minimum(j, 255)
        excl = plsc.load_gather(gbl_v, [jnp.maximum(j - 1, 0)])
        t3 = t - jnp.where(j > 0, excl, jnp.float32(0.0))
        pbase = (j >> 4) * _CHUNK + (j & 15)
        lo = jnp.zeros((16,), jnp.int32)
        for b in (256, 128, 64, 32, 16, 8, 4, 2, 1):
            probe = jnp.minimum(lo + (b - 1), _SEG - 1)
            val = plsc.load_gather(cdf_v, [pbase + (probe << 4)])
            lo = lo + jnp.where(val < t3, b, 0)
        return jnp.minimum(j * _SEG + lo, _V - 1)

    with jax.named_scope("phase_search"):
        def grp_body(g, _):
            for d in range(_UNROLL_G):
                gg = g * _UNROLL_G + d
                t = u_v[pl.ds(gg * 16, 16)] * total
                idx = search16(t)
                o_v[pl.ds(gg * 16, 16)] = idx.astype(jnp.float32)
            return 0

        lax.fori_loop(0, _GRPS // _UNROLL_G, grp_body, 0)

    pltpu.sync_copy(o_v, out_hbm.at[pl.ds(wid * _SAMP_W, _SAMP_W)])


def kernel(data, freqs):
    del data
    u = jax.random.uniform(jax.random.key(42), (_NSAMP,), dtype=jnp.float32)
    mesh = plsc.VectorSubcoreMesh(core_axis_name="c", subcore_axis_name="s")
    run = pl.kernel(
        _sampler,
        mesh=mesh,
        compiler_params=pltpu.CompilerParams(needs_layout_passes=False),
        out_type=jax.ShapeDtypeStruct((_NSAMP,), jnp.float32),
        scratch_types=[
            pltpu.VMEM((_NPAD,), jnp.float32),
            pltpu.VMEM((_CHUNK,), jnp.float32),
            pltpu.VMEM((_SAMP_W,), jnp.float32),
            pltpu.VMEM((_SAMP_W,), jnp.float32),
            pltpu.VMEM((256,), jnp.float32),
            pltpu.VMEM_SHARED((_NPAD,), jnp.float32),
            pltpu.SemaphoreType.DMA,
            pltpu.SemaphoreType.DMA,
        ],
    )
    return run(freqs, u)


# speedup vs baseline: 1.2340x; 1.2161x over previous
import functools

import numpy as np
import jax
import jax.numpy as jnp
from jax import lax
from jax.experimental import pallas as pl
from jax.experimental.pallas import tpu as pltpu
from jax.experimental.pallas import tpu_sc as plsc

_NSAMP = 16384
_V = 100000
_NPAD = 100352
_CHUNK = _NPAD // 16
_SEG = _CHUNK // 16
_TAIL = _V - 15 * _CHUNK
_SAMP_W = _NSAMP // 32
_GRPS = _SAMP_W // 16
_UNROLL_G = 4
_UNROLL_K = 8

def _lane_gather(v, idx):
    dnums = lax.GatherDimensionNumbers(
        offset_dims=(), collapsed_slice_dims=(0,), start_index_map=(0,))
    return lax.gather(v, idx[:, None], dnums, slice_sizes=(1,),
                      mode=lax.GatherScatterMode.PROMISE_IN_BOUNDS)


def _lane_prefix(v, lanes):
    for k in (1, 2, 4, 8):
        sh = _lane_gather(v, jnp.maximum(lanes - k, 0))
        v = v + jnp.where(lanes >= k, sh, jnp.float32(0.0))
    return v


def _sampler(freqs_hbm, u_hbm, out_hbm,
             cdf_v, fr_v, u_v, o_v, gbl_v, cdf_sh, sem_f, sem_u):
    c_id = lax.axis_index("c")
    s_id = lax.axis_index("s")
    wid = s_id * 2 + c_id
    base = s_id * _CHUNK
    lanes = lax.iota(jnp.int32, 16)

    h_f = pltpu.async_copy(freqs_hbm.at[pl.ds(base, _TAIL)],
                           fr_v.at[pl.ds(0, _TAIL)], sem_f)
    h_u = pltpu.async_copy(u_hbm.at[pl.ds(wid * _SAMP_W, _SAMP_W)], u_v, sem_u)

    @pl.when(s_id == 15)
    def _():
        for z in range((_CHUNK - _TAIL) // 16):
            fr_v[pl.ds(_TAIL + z * 16, 16)] = jnp.zeros((16,), jnp.float32)

    @pl.when(s_id != 15)
    def _():
        pltpu.sync_copy(freqs_hbm.at[pl.ds(base + _TAIL, _CHUNK - _TAIL)],
                        fr_v.at[pl.ds(_TAIL, _CHUNK - _TAIL)])

    h_f.wait()

    with jax.named_scope("phase_cumsum"):
        rd_base = lanes * _SEG

        @plsc.parallel_loop(0, _SEG, 1, unroll=_UNROLL_K,
                            carry=jnp.zeros((16,), jnp.float32))
        def _pass_a(k, acc):
            acc = acc + plsc.load_gather(fr_v, [rd_base + k])
            cdf_v[pl.ds(base + k * 16, 16)] = acc
            return acc

    with jax.named_scope("phase_exchange"):
        pltpu.sync_copy(cdf_v.at[pl.ds(base, _CHUNK)],
                        cdf_sh.at[pl.ds(base, _CHUNK)])
        plsc.subcore_barrier()
        pltpu.sync_copy(cdf_sh, cdf_v)

    with jax.named_scope("phase_tables"):
        for r in range(16):
            srow = cdf_v[pl.ds(r * _CHUNK + (_SEG - 1) * 16, 16)]
            gbl_v[pl.ds(r * 16, 16)] = _lane_prefix(srow, lanes)
        tot_c = plsc.load_gather(gbl_v, [lanes * 16 + 15])
        bnd = _lane_prefix(tot_c, lanes)
        off = bnd - tot_c
        total = _lane_gather(bnd, jnp.full((16,), 15, jnp.int32))
        for r in range(16):
            offr = _lane_gather(off, jnp.full((16,), r, jnp.int32))
            gbl_v[pl.ds(r * 16, 16)] = gbl_v[pl.ds(r * 16, 16)] + offr

    h_u.wait()

    def search16(t):
        j = jnp.zeros((16,), jnp.int32)
        for b in (128, 64, 32, 16, 8, 4, 2, 1):
            val = plsc.load_gather(gbl_v, [j + (b - 1)])
            j = j + jnp.where(val < t, b, 0)
        j = jnp.minimum(j, 255)
        excl = plsc.load_gather(gbl_v, [jnp.maximum(j - 1, 0)])
        t3 = t - jnp.where(j > 0, excl, jnp.float32(0.0))
        pbase = (j >> 4) * _CHUNK + (j & 15)
        lo = jnp.zeros((16,), jnp.int32)
        for b in (256, 128, 64, 32, 16, 8, 4, 2, 1):
            probe = jnp.minimum(lo + (b - 1), _SEG - 1)
            val = plsc.load_gather(cdf_v, [pbase + (probe << 4)])
            lo = lo + jnp.where(val < t3, b, 0)
        return jnp.minimum(j * _SEG + lo, _V - 1)

    with jax.named_scope("phase_search"):
        @plsc.parallel_loop(0, _GRPS, 1, unroll=_UNROLL_G)
        def _grp_body(gg):
            t = u_v[pl.ds(gg * 16, 16)] * total
            idx = search16(t)
            o_v[pl.ds(gg * 16, 16)] = idx.astype(jnp.float32)

    pltpu.sync_copy(o_v, out_hbm.at[pl.ds(wid * _SAMP_W, _SAMP_W)])


def kernel(data, freqs):
    del data
    u = jax.random.uniform(jax.random.key(42), (_NSAMP,), dtype=jnp.float32)
    mesh = plsc.VectorSubcoreMesh(core_axis_name="c", subcore_axis_name="s")
    run = pl.kernel(
        _sampler,
        mesh=mesh,
        compiler_params=pltpu.CompilerParams(needs_layout_passes=False),
        out_type=jax.ShapeDtypeStruct((_NSAMP,), jnp.float32),
        scratch_types=[
            pltpu.VMEM((_NPAD,), jnp.float32),
            pltpu.VMEM((_CHUNK,), jnp.float32),
            pltpu.VMEM((_SAMP_W,), jnp.float32),
            pltpu.VMEM((_SAMP_W,), jnp.float32),
            pltpu.VMEM((256,), jnp.float32),
            pltpu.VMEM_SHARED((_NPAD,), jnp.float32),
            pltpu.SemaphoreType.DMA,
            pltpu.SemaphoreType.DMA,
        ],
    )
    return run(freqs, u)
